# baseline (speedup 1.0000x reference)
import dataclasses

import jax
import jax.numpy as jnp
from jax import lax
from jax.experimental import pallas as pl
from jax.experimental.pallas import tpu as pltpu
from jax.experimental.pallas import tpu_sc as plsc

N_NODES = 10000
NPAD = 10240
DIM = 128
DE = 16
E_TOTAL = 320000
NSC = 2
NSUB = 16
HDIM = DIM // NSC
EPW = E_TOTAL // NSUB
C = 128
CT = 32
RPS = NPAD // NSUB

_HI = lax.Precision.HIGHEST


def _prep_body(x_ref, ts_ref, w1_ref, w2_ref, bt_ref, fr_ref, ph_ref, h_ref):
    t_enc = jnp.cos(ts_ref[...] * fr_ref[...] + ph_ref[...])
    h = jnp.dot(x_ref[...], w1_ref[...], preferred_element_type=jnp.float32,
                precision=_HI)
    h += jnp.dot(t_enc, w2_ref[...], preferred_element_type=jnp.float32,
                 precision=_HI)
    h_ref[...] = h + bt_ref[...]


def _rst_base_body(h_ref, ws_ref, bs_ref, bn_ref, o_ref):
    o_ref[...] = (jnp.dot(h_ref[...], ws_ref[...],
                          preferred_element_type=jnp.float32, precision=_HI)
                  + bs_ref[...] + bn_ref[...])


def _combine_body(acc_ref, aux_ref, deg_ref, ones_ref, we_ref, be_ref,
                  wn_ref, rb_ref, o_ref, carry_ref):
    i = pl.program_id(0)

    @pl.when(i == 0)
    def _():
        carry_ref[...] = jnp.zeros((1, DIM), jnp.float32)

    a = jnp.concatenate([acc_ref[0], acc_ref[1]], axis=1)
    s = aux_ref[...]
    deg = lax.dot_general(deg_ref[...], ones_ref[...],
                          (((0,), (0,)), ((), ())),
                          preferred_element_type=jnp.float32,
                          precision=_HI)
    h_ns = a + jnp.dot(s, we_ref[...], preferred_element_type=jnp.float32,
                       precision=_HI) + deg * be_ref[...]
    b = h_ns.shape[0]
    r = lax.broadcasted_iota(jnp.int32, (b, b), 0)
    c = lax.broadcasted_iota(jnp.int32, (b, b), 1)
    tril = (r >= c).astype(jnp.float32)
    cs = jnp.dot(tril, h_ns, preferred_element_type=jnp.float32,
                 precision=_HI) + carry_ref[...]
    carry_ref[...] = cs[b - 1:b, :]
    h_neigh = cs / jnp.maximum(deg, 1.0)
    o_ref[...] = rb_ref[...] + jnp.dot(h_neigh, wn_ref[...],
                                       preferred_element_type=jnp.float32,
                                       precision=_HI)


def _sc_body(hlo_hbm, hhi_hbm, src_hbm, dst_hbm, attr_hbm,
             acc_out, aux_out, deg_out,
             src_v, dst_v, src_t, dst_t, rows_v, rows_t, attr_v, attr_t,
             deg_v, acc_s, aux_s, semi0, semi1, semg0, semg1, sems0, sems1,
             semt):
    core = lax.axis_index("c")
    sub = lax.axis_index("s")
    z16 = jnp.zeros((16,), jnp.float32)
    o16 = jnp.ones((16,), jnp.float32)


    @pl.loop(0, C)
    def _(r):
        @pl.loop(0, HDIM, step=16)
        def _(j):
            rows_v[0, r, pl.ds(j, 16)] = z16

    @pl.loop(0, C)
    def _(r):
        attr_v[0, r, pl.ds(0, 16)] = z16

    @pl.loop(0, NPAD, step=16)
    def _(r):
        deg_v[pl.ds(r, 16)] = z16

    rbase = sub * RPS

    @pl.loop(0, RPS, step=C)
    def _(k):
        pltpu.sync_copy(rows_v.at[0], acc_s.at[pl.ds(rbase + k, C)])
        pltpu.sync_copy(attr_v.at[0], aux_s.at[pl.ds(rbase + k, C)])

    plsc.subcore_barrier()

    ebase = sub * EPW
    nmain = EPW - CT

    semi = (semi0, semi1)
    semg = (semg0, semg1)
    sems = (sems0, sems1)

    def idx_dmas(i, b):
        yield pltpu.make_async_copy(src_hbm.at[pl.ds(ebase + i, C)],
                                    src_v.at[b], semi[b])
        yield pltpu.make_async_copy(dst_hbm.at[pl.ds(ebase + i, C)],
                                    dst_v.at[b], semi[b])

    def attr_dma(i, b):
        return pltpu.make_async_copy(attr_hbm.at[pl.ds(ebase + i, C)],
                                     attr_v.at[b], semi[b])

    def idx_issue(i, b):
        for d in idx_dmas(i, b):
            d.start()

        @pl.when(core == 0)
        def _():
            attr_dma(i, b).start()

    def idx_wait(i, b):
        for d in idx_dmas(i, b):
            d.wait()

        @pl.when(core == 0)
        def _():
            attr_dma(i, b).wait()

    def gather_start(b):
        @pl.when(core == 0)
        def _():
            pltpu.make_async_copy(hlo_hbm.at[src_v.at[b]], rows_v.at[b],
                                  semg[b]).start()

        @pl.when(core == 1)
        def _():
            pltpu.make_async_copy(hhi_hbm.at[src_v.at[b]], rows_v.at[b],
                                  semg[b]).start()

    def gather_wait(b):
        pltpu.make_async_copy(hlo_hbm.at[src_v.at[b]], rows_v.at[b],
                              semg[b]).wait()

    def deg_update(b):
        @pl.when(core == 1)
        def _():
            for j in range(C // 16):
                idx = dst_v[b, pl.ds(j * 16, 16)]
                plsc.addupdate_scatter(deg_v, [idx], o16)

    def scatter_issue(b):
        pltpu.make_async_copy(rows_v.at[b], acc_s.at[dst_v.at[b]],
                              sems[b]).start(add=True)

        @pl.when(core == 0)
        def _():
            pltpu.make_async_copy(attr_v.at[b], aux_s.at[dst_v.at[b]],
                                  sems[b]).start(add=True)

    def scatter_wait(b):
        pltpu.make_async_copy(rows_v.at[b], acc_s.at[dst_v.at[b]],
                              sems[b]).wait()

        @pl.when(core == 0)
        def _():
            pltpu.make_async_copy(attr_v.at[b], aux_s.at[dst_v.at[b]],
                                  sems[b]).wait()

    idx_issue(0, 0)
    idx_issue(C, 1)

    @pl.loop(0, nmain, step=2 * C)
    def _(i):
        idx_wait(i, 0)
        gather_start(0)
        deg_update(0)
        idx_wait(i + C, 1)
        gather_start(1)
        deg_update(1)
        gather_wait(0)
        scatter_issue(0)
        gather_wait(1)
        scatter_issue(1)
        scatter_wait(0)

        @pl.when(i + 2 * C < nmain)
        def _():
            idx_issue(i + 2 * C, 0)

        scatter_wait(1)

        @pl.when(i + 3 * C < nmain)
        def _():
            idx_issue(i + 3 * C, 1)

    bt = ebase + nmain
    pltpu.sync_copy(src_hbm.at[pl.ds(bt, CT)], src_t)
    pltpu.sync_copy(dst_hbm.at[pl.ds(bt, CT)], dst_t)

    @pl.when(core == 0)
    def _():
        pltpu.async_copy(hlo_hbm.at[src_t], rows_t, semt).wait()
        pltpu.sync_copy(rows_t, acc_s.at[dst_t], add=True)
        pltpu.sync_copy(attr_hbm.at[pl.ds(bt, CT)], attr_t)
        pltpu.sync_copy(attr_t, aux_s.at[dst_t], add=True)

    @pl.when(core == 1)
    def _():
        pltpu.async_copy(hhi_hbm.at[src_t], rows_t, semt).wait()
        pltpu.sync_copy(rows_t, acc_s.at[dst_t], add=True)
        for j in range(CT // 16):
            idx = dst_t[pl.ds(j * 16, 16)]
            plsc.addupdate_scatter(deg_v, [idx], o16)

    plsc.subcore_barrier()

    pltpu.sync_copy(acc_s.at[pl.ds(rbase, RPS)],
                    acc_out.at[core, pl.ds(rbase, RPS)])

    @pl.when(core == 0)
    def _():
        pltpu.sync_copy(aux_s.at[pl.ds(rbase, RPS)],
                        aux_out.at[pl.ds(rbase, RPS)])

    @pl.when(core == 1)
    def _():
        pltpu.sync_copy(deg_v, deg_out.at[sub])


def _full(u):
    return pl.BlockSpec(u, lambda *_: tuple(0 for _ in u))


def _sc_compiler_params():
    cp = pltpu.CompilerParams(use_tc_tiling_on_sc=False)
    if "needs_layout_passes" in pltpu.CompilerParams.__dataclass_fields__:
        cp = dataclasses.replace(cp, needs_layout_passes=False)
    return cp


def kernel(x, timestamps, edge_index, edge_attr, new_node_ids,
           time_freq, time_phase, W_time, b_time,
           W_edge, b_edge, W_self, b_self, W_neigh, b_neigh):
    del new_node_ids

    f32 = jnp.float32
    npad = NPAD - N_NODES
    x_p = jnp.pad(x, ((0, npad), (0, 0)))
    ts_p = jnp.pad(jnp.broadcast_to(timestamps[:, None], (N_NODES, DIM)),
                   ((0, npad), (0, 0)))
    src = edge_index[0]
    dst = edge_index[1]
    w1 = W_time[:DIM]
    w2 = W_time[DIM:]

    blk = 1024
    grid1 = NPAD // blk
    h_self = pl.pallas_call(
        _prep_body,
        grid=(grid1,),
        in_specs=[
            pl.BlockSpec((blk, DIM), lambda i: (i, 0)),
            pl.BlockSpec((blk, DIM), lambda i: (i, 0)),
            _full((DIM, DIM)), _full((DIM, DIM)),
            _full((1, DIM)), _full((1, DIM)), _full((1, DIM)),
        ],
        out_specs=pl.BlockSpec((blk, DIM), lambda i: (i, 0)),
        out_shape=jax.ShapeDtypeStruct((NPAD, DIM), f32),
    )(x_p, ts_p, w1, w2, b_time[None, :], time_freq[None, :],
      time_phase[None, :])

    h_lo = h_self[:, :HDIM]
    h_hi = h_self[:, HDIM:]

    mesh = plsc.VectorSubcoreMesh(core_axis_name="c", subcore_axis_name="s")
    sc_fn = pl.kernel(
        _sc_body,
        out_type=(
            jax.ShapeDtypeStruct((NSC, NPAD, HDIM), f32),
            jax.ShapeDtypeStruct((NPAD, DE), f32),
            jax.ShapeDtypeStruct((NSUB, NPAD), f32),
        ),
        mesh=mesh,
        compiler_params=_sc_compiler_params(),
        scratch_types=[
            pltpu.VMEM((2, C), jnp.int32),
            pltpu.VMEM((2, C), jnp.int32),
            pltpu.VMEM((CT,), jnp.int32),
            pltpu.VMEM((CT,), jnp.int32),
            pltpu.VMEM((2, C, HDIM), f32),
            pltpu.VMEM((CT, HDIM), f32),
            pltpu.VMEM((2, C, DE), f32),
            pltpu.VMEM((CT, DE), f32),
            pltpu.VMEM((NPAD,), f32),
            pltpu.VMEM_SHARED((NPAD, HDIM), f32),
            pltpu.VMEM_SHARED((NPAD, DE), f32),
            pltpu.SemaphoreType.DMA,
            pltpu.SemaphoreType.DMA,
            pltpu.SemaphoreType.DMA,
            pltpu.SemaphoreType.DMA,
            pltpu.SemaphoreType.DMA,
            pltpu.SemaphoreType.DMA,
            pltpu.SemaphoreType.DMA,
        ],
    )
    acc, aux, dega = sc_fn(h_lo, h_hi, src, dst, edge_attr)

    rst_base = pl.pallas_call(
        _rst_base_body,
        grid=(grid1,),
        in_specs=[
            pl.BlockSpec((blk, DIM), lambda i: (i, 0)),
            _full((DIM, DIM)), _full((1, DIM)), _full((1, DIM)),
        ],
        out_specs=pl.BlockSpec((blk, DIM), lambda i: (i, 0)),
        out_shape=jax.ShapeDtypeStruct((NPAD, DIM), f32),
    )(h_self, W_self, b_self[None, :], b_neigh[None, :])

    cblk = 256
    grid4 = NPAD // cblk
    rst = pl.pallas_call(
        _combine_body,
        grid=(grid4,),
        in_specs=[
            pl.BlockSpec((NSC, cblk, HDIM), lambda i: (0, i, 0)),
            pl.BlockSpec((cblk, DE), lambda i: (i, 0)),
            pl.BlockSpec((NSUB, cblk), lambda i: (0, i)),
            _full((NSUB, 1)),
            _full((DE, DIM)), _full((1, DIM)), _full((DIM, DIM)),
            pl.BlockSpec((cblk, DIM), lambda i: (i, 0)),
        ],
        out_specs=pl.BlockSpec((cblk, DIM), lambda i: (i, 0)),
        out_shape=jax.ShapeDtypeStruct((NPAD, DIM), f32),
        scratch_shapes=[pltpu.VMEM((1, DIM), f32)],
    )(acc, aux, dega, jnp.ones((NSUB, 1), f32), W_edge, b_edge[None, :],
      W_neigh, rst_base)

    return rst[:N_NODES]


# speedup vs baseline: 1.2653x; 1.2653x over previous
import dataclasses

import jax
import jax.numpy as jnp
from jax import lax
from jax.experimental import pallas as pl
from jax.experimental.pallas import tpu as pltpu
from jax.experimental.pallas import tpu_sc as plsc

N_NODES = 10000
NPAD = 10240
DIM = 128
DE = 16
E_TOTAL = 320000
NSC = 2
NSUB = 16
HDIM = DIM // NSC
EPW = E_TOTAL // NSUB
C = 128
CT = 32
EPW_B = E_TOTAL // (NSC * NSUB)
CT_B = 16
RPS = NPAD // NSUB


def _prep_body(x_ref, ts_ref, w1_ref, w2_ref, bt_ref, fr_ref, ph_ref, h_ref):
    t_enc = jnp.cos(ts_ref[...] * fr_ref[...] + ph_ref[...])
    h = jnp.dot(x_ref[...], w1_ref[...], preferred_element_type=jnp.float32)
    h += jnp.dot(t_enc, w2_ref[...], preferred_element_type=jnp.float32)
    h_ref[...] = h + bt_ref[...]


def _rst_base_body(h_ref, ws_ref, bs_ref, bn_ref, o_ref):
    o_ref[...] = (jnp.dot(h_ref[...], ws_ref[...],
                          preferred_element_type=jnp.float32)
                  + bs_ref[...] + bn_ref[...])


def _combine_body(acc_ref, aux_ref, deg_ref, ones_ref, we_ref, be_ref,
                  wn_ref, rb_ref, o_ref, carry_ref):
    i = pl.program_id(0)

    @pl.when(i == 0)
    def _():
        carry_ref[...] = jnp.zeros((1, DIM), jnp.float32)

    a = jnp.concatenate([acc_ref[0], acc_ref[1]], axis=1)
    s = aux_ref[0] + aux_ref[1]
    deg = lax.dot_general(deg_ref[...], ones_ref[...],
                          (((0,), (0,)), ((), ())),
                          preferred_element_type=jnp.float32)
    h_ns = a + jnp.dot(s, we_ref[...], preferred_element_type=jnp.float32) + deg * be_ref[...]
    b = h_ns.shape[0]
    r = lax.broadcasted_iota(jnp.int32, (b, b), 0)
    c = lax.broadcasted_iota(jnp.int32, (b, b), 1)
    tril = (r >= c).astype(jnp.float32)
    cs = jnp.dot(tril, h_ns, preferred_element_type=jnp.float32) + carry_ref[...]
    carry_ref[...] = cs[b - 1:b, :]
    h_neigh = cs / jnp.maximum(deg, 1.0)
    o_ref[...] = rb_ref[...] + jnp.dot(h_neigh, wn_ref[...],
                                       preferred_element_type=jnp.float32)


def _sc_a_body(hlo_hbm, hhi_hbm, src_hbm, dst_hbm,
               acc_out, deg_out,
               src_v, dst_v, src_t, dst_t, rows_v, rows_t,
               deg_v, acc_s, semi0, semi1, semg0, semg1, sems0, sems1,
               semt):
    core = lax.axis_index("c")
    sub = lax.axis_index("s")
    z16 = jnp.zeros((16,), jnp.float32)
    o16 = jnp.ones((16,), jnp.float32)


    @pl.loop(0, C)
    def _(r):
        @pl.loop(0, HDIM, step=16)
        def _(j):
            rows_v[0, r, pl.ds(j, 16)] = z16

    @pl.loop(0, NPAD, step=16)
    def _(r):
        deg_v[pl.ds(r, 16)] = z16

    rbase = sub * RPS

    @pl.loop(0, RPS, step=C)
    def _(k):
        pltpu.sync_copy(rows_v.at[0], acc_s.at[pl.ds(rbase + k, C)])

    plsc.subcore_barrier()

    ebase = sub * EPW
    nmain = EPW - CT

    semi = (semi0, semi1)
    semg = (semg0, semg1)
    sems = (sems0, sems1)

    def idx_dmas(i, b):
        yield pltpu.make_async_copy(src_hbm.at[pl.ds(ebase + i, C)],
                                    src_v.at[b], semi[b])
        yield pltpu.make_async_copy(dst_hbm.at[pl.ds(ebase + i, C)],
                                    dst_v.at[b], semi[b])

    def idx_issue(i, b):
        for d in idx_dmas(i, b):
            d.start()

    def idx_wait(i, b):
        for d in idx_dmas(i, b):
            d.wait()

    def gather_start(b):
        @pl.when(core == 0)
        def _():
            pltpu.make_async_copy(hlo_hbm.at[src_v.at[b]], rows_v.at[b],
                                  semg[b]).start()

        @pl.when(core == 1)
        def _():
            pltpu.make_async_copy(hhi_hbm.at[src_v.at[b]], rows_v.at[b],
                                  semg[b]).start()

    def gather_wait(b):
        pltpu.make_async_copy(hlo_hbm.at[src_v.at[b]], rows_v.at[b],
                              semg[b]).wait()

    def deg_update(b):
        @pl.when(core == 1)
        def _():
            for j in range(C // 16):
                idx = dst_v[b, pl.ds(j * 16, 16)]
                plsc.addupdate_scatter(deg_v, [idx], o16)

    def scatter_issue(b):
        pltpu.make_async_copy(rows_v.at[b], acc_s.at[dst_v.at[b]],
                              sems[b]).start(add=True)

    def scatter_wait(b):
        pltpu.make_async_copy(rows_v.at[b], acc_s.at[dst_v.at[b]],
                              sems[b]).wait()

    idx_issue(0, 0)
    idx_issue(C, 1)

    @pl.loop(0, nmain, step=2 * C)
    def _(i):
        idx_wait(i, 0)
        gather_start(0)
        deg_update(0)
        idx_wait(i + C, 1)
        gather_start(1)
        deg_update(1)
        gather_wait(0)
        scatter_issue(0)
        gather_wait(1)
        scatter_issue(1)
        scatter_wait(0)

        @pl.when(i + 2 * C < nmain)
        def _():
            idx_issue(i + 2 * C, 0)

        scatter_wait(1)

        @pl.when(i + 3 * C < nmain)
        def _():
            idx_issue(i + 3 * C, 1)

    bt = ebase + nmain
    pltpu.sync_copy(src_hbm.at[pl.ds(bt, CT)], src_t)
    pltpu.sync_copy(dst_hbm.at[pl.ds(bt, CT)], dst_t)

    @pl.when(core == 0)
    def _():
        pltpu.async_copy(hlo_hbm.at[src_t], rows_t, semt).wait()
        pltpu.sync_copy(rows_t, acc_s.at[dst_t], add=True)

    @pl.when(core == 1)
    def _():
        pltpu.async_copy(hhi_hbm.at[src_t], rows_t, semt).wait()
        pltpu.sync_copy(rows_t, acc_s.at[dst_t], add=True)
        for j in range(CT // 16):
            idx = dst_t[pl.ds(j * 16, 16)]
            plsc.addupdate_scatter(deg_v, [idx], o16)

    plsc.subcore_barrier()

    pltpu.sync_copy(acc_s.at[pl.ds(rbase, RPS)],
                    acc_out.at[core, pl.ds(rbase, RPS)])

    @pl.when(core == 1)
    def _():
        pltpu.sync_copy(deg_v, deg_out.at[sub])


def _sc_b_body(dst_hbm, attr_hbm, aux_out,
               dst_v, dst_t, attr_v, attr_t, aux_s,
               semi0, semi1, sems0, sems1, semt):
    core = lax.axis_index("c")
    sub = lax.axis_index("s")
    z16 = jnp.zeros((16,), jnp.float32)

    @pl.loop(0, C)
    def _(r):
        attr_v[0, r, pl.ds(0, 16)] = z16

    rbase = sub * RPS

    @pl.loop(0, RPS, step=C)
    def _(k):
        pltpu.sync_copy(attr_v.at[0], aux_s.at[pl.ds(rbase + k, C)])

    plsc.subcore_barrier()

    ebase = (core * NSUB + sub) * EPW_B
    nmain = EPW_B - CT_B

    semi = (semi0, semi1)
    sems = (sems0, sems1)

    def in_dmas(i, b):
        yield pltpu.make_async_copy(dst_hbm.at[pl.ds(ebase + i, C)],
                                    dst_v.at[b], semi[b])
        yield pltpu.make_async_copy(attr_hbm.at[pl.ds(ebase + i, C)],
                                    attr_v.at[b], semi[b])

    def in_issue(i, b):
        for d in in_dmas(i, b):
            d.start()

    def in_wait(i, b):
        for d in in_dmas(i, b):
            d.wait()

    def scatter(b, start):
        d = pltpu.make_async_copy(attr_v.at[b], aux_s.at[dst_v.at[b]],
                                  sems[b])
        if start:
            d.start(add=True)
        else:
            d.wait()

    in_issue(0, 0)
    in_issue(C, 1)

    @pl.loop(0, nmain, step=2 * C)
    def _(i):
        in_wait(i, 0)
        scatter(0, True)
        in_wait(i + C, 1)
        scatter(1, True)
        scatter(0, False)

        @pl.when(i + 2 * C < nmain)
        def _():
            in_issue(i + 2 * C, 0)

        scatter(1, False)

        @pl.when(i + 3 * C < nmain)
        def _():
            in_issue(i + 3 * C, 1)

    bt = ebase + nmain
    pltpu.sync_copy(dst_hbm.at[pl.ds(bt, CT_B)], dst_t)
    pltpu.sync_copy(attr_hbm.at[pl.ds(bt, CT_B)], attr_t)
    pltpu.sync_copy(attr_t, aux_s.at[dst_t], add=True)
    del semt

    plsc.subcore_barrier()

    pltpu.sync_copy(aux_s.at[pl.ds(rbase, RPS)],
                    aux_out.at[core, pl.ds(rbase, RPS)])


def _full(u):
    return pl.BlockSpec(u, lambda *_: tuple(0 for _ in u))


def _sc_compiler_params():
    cp = pltpu.CompilerParams(use_tc_tiling_on_sc=False)
    if "needs_layout_passes" in pltpu.CompilerParams.__dataclass_fields__:
        cp = dataclasses.replace(cp, needs_layout_passes=False)
    return cp


def kernel(x, timestamps, edge_index, edge_attr, new_node_ids,
           time_freq, time_phase, W_time, b_time,
           W_edge, b_edge, W_self, b_self, W_neigh, b_neigh):
    del new_node_ids

    f32 = jnp.float32
    npad = NPAD - N_NODES
    x_p = jnp.pad(x, ((0, npad), (0, 0)))
    ts_p = jnp.pad(jnp.broadcast_to(timestamps[:, None], (N_NODES, DIM)),
                   ((0, npad), (0, 0)))
    src = edge_index[0]
    dst = edge_index[1]
    w1 = W_time[:DIM]
    w2 = W_time[DIM:]

    blk = 1024
    grid1 = NPAD // blk
    h_self = pl.pallas_call(
        _prep_body,
        grid=(grid1,),
        in_specs=[
            pl.BlockSpec((blk, DIM), lambda i: (i, 0)),
            pl.BlockSpec((blk, DIM), lambda i: (i, 0)),
            _full((DIM, DIM)), _full((DIM, DIM)),
            _full((1, DIM)), _full((1, DIM)), _full((1, DIM)),
        ],
        out_specs=pl.BlockSpec((blk, DIM), lambda i: (i, 0)),
        out_shape=jax.ShapeDtypeStruct((NPAD, DIM), f32),
    )(x_p, ts_p, w1, w2, b_time[None, :], time_freq[None, :],
      time_phase[None, :])

    h_lo = h_self[:, :HDIM]
    h_hi = h_self[:, HDIM:]

    mesh = plsc.VectorSubcoreMesh(core_axis_name="c", subcore_axis_name="s")
    sc_a = pl.kernel(
        _sc_a_body,
        out_type=(
            jax.ShapeDtypeStruct((NSC, NPAD, HDIM), f32),
            jax.ShapeDtypeStruct((NSUB, NPAD), f32),
        ),
        mesh=mesh,
        compiler_params=_sc_compiler_params(),
        scratch_types=[
            pltpu.VMEM((2, C), jnp.int32),
            pltpu.VMEM((2, C), jnp.int32),
            pltpu.VMEM((CT,), jnp.int32),
            pltpu.VMEM((CT,), jnp.int32),
            pltpu.VMEM((2, C, HDIM), f32),
            pltpu.VMEM((CT, HDIM), f32),
            pltpu.VMEM((NPAD,), f32),
            pltpu.VMEM_SHARED((NPAD, HDIM), f32),
            pltpu.SemaphoreType.DMA,
            pltpu.SemaphoreType.DMA,
            pltpu.SemaphoreType.DMA,
            pltpu.SemaphoreType.DMA,
            pltpu.SemaphoreType.DMA,
            pltpu.SemaphoreType.DMA,
            pltpu.SemaphoreType.DMA,
        ],
    )
    acc, dega = sc_a(h_lo, h_hi, src, dst)

    sc_b = pl.kernel(
        _sc_b_body,
        out_type=jax.ShapeDtypeStruct((NSC, NPAD, DE), f32),
        mesh=mesh,
        compiler_params=_sc_compiler_params(),
        scratch_types=[
            pltpu.VMEM((2, C), jnp.int32),
            pltpu.VMEM((CT_B,), jnp.int32),
            pltpu.VMEM((2, C, DE), f32),
            pltpu.VMEM((CT_B, DE), f32),
            pltpu.VMEM_SHARED((NPAD, DE), f32),
            pltpu.SemaphoreType.DMA,
            pltpu.SemaphoreType.DMA,
            pltpu.SemaphoreType.DMA,
            pltpu.SemaphoreType.DMA,
            pltpu.SemaphoreType.DMA,
        ],
    )
    aux = sc_b(dst, edge_attr)

    rst_base = pl.pallas_call(
        _rst_base_body,
        grid=(grid1,),
        in_specs=[
            pl.BlockSpec((blk, DIM), lambda i: (i, 0)),
            _full((DIM, DIM)), _full((1, DIM)), _full((1, DIM)),
        ],
        out_specs=pl.BlockSpec((blk, DIM), lambda i: (i, 0)),
        out_shape=jax.ShapeDtypeStruct((NPAD, DIM), f32),
    )(h_self, W_self, b_self[None, :], b_neigh[None, :])

    cblk = 256
    grid4 = NPAD // cblk
    rst = pl.pallas_call(
        _combine_body,
        grid=(grid4,),
        in_specs=[
            pl.BlockSpec((NSC, cblk, HDIM), lambda i: (0, i, 0)),
            pl.BlockSpec((NSC, cblk, DE), lambda i: (0, i, 0)),
            pl.BlockSpec((NSUB, cblk), lambda i: (0, i)),
            _full((NSUB, 1)),
            _full((DE, DIM)), _full((1, DIM)), _full((DIM, DIM)),
            pl.BlockSpec((cblk, DIM), lambda i: (i, 0)),
        ],
        out_specs=pl.BlockSpec((cblk, DIM), lambda i: (i, 0)),
        out_shape=jax.ShapeDtypeStruct((NPAD, DIM), f32),
        scratch_shapes=[pltpu.VMEM((1, DIM), f32)],
    )(acc, aux, dega, jnp.ones((NSUB, 1), f32), W_edge, b_edge[None, :],
      W_neigh, rst_base)

    return rst[:N_NODES]


# speedup vs baseline: 1.5172x; 1.1991x over previous
import dataclasses

import jax
import jax.numpy as jnp
from jax import lax
from jax.experimental import pallas as pl
from jax.experimental.pallas import tpu as pltpu
from jax.experimental.pallas import tpu_sc as plsc

N_NODES = 10000
NPAD = 10240
DIM = 128
DE = 16
E_TOTAL = 320000
NSC = 2
NSUB = 16
HDIM = DIM // NSC
EPW = E_TOTAL // NSUB
C = 128
NB = 4
CT = 32
EPW_B = E_TOTAL // (NSC * NSUB)
CT_B = 16
RPS = NPAD // NSUB


def _prep_body(x_ref, ts_ref, w1_ref, w2_ref, bt_ref, fr_ref, ph_ref, h_ref):
    t_enc = jnp.cos(ts_ref[...] * fr_ref[...] + ph_ref[...])
    h = jnp.dot(x_ref[...], w1_ref[...], preferred_element_type=jnp.float32)
    h += jnp.dot(t_enc, w2_ref[...], preferred_element_type=jnp.float32)
    h_ref[...] = h + bt_ref[...]


def _rst_base_body(h_ref, ws_ref, bs_ref, bn_ref, o_ref):
    o_ref[...] = (jnp.dot(h_ref[...], ws_ref[...],
                          preferred_element_type=jnp.float32)
                  + bs_ref[...] + bn_ref[...])


def _combine_body(acc_ref, aux_ref, deg_ref, ones_ref, we_ref, be_ref,
                  wn_ref, rb_ref, o_ref, carry_ref):
    i = pl.program_id(0)

    @pl.when(i == 0)
    def _():
        carry_ref[...] = jnp.zeros((1, DIM), jnp.float32)

    a = jnp.concatenate([acc_ref[0], acc_ref[1]], axis=1)
    s = aux_ref[0] + aux_ref[1]
    deg = lax.dot_general(deg_ref[...], ones_ref[...],
                          (((0,), (0,)), ((), ())),
                          preferred_element_type=jnp.float32)
    h_ns = a + jnp.dot(s, we_ref[...], preferred_element_type=jnp.float32) + deg * be_ref[...]
    b = h_ns.shape[0]
    r = lax.broadcasted_iota(jnp.int32, (b, b), 0)
    c = lax.broadcasted_iota(jnp.int32, (b, b), 1)
    tril = (r >= c).astype(jnp.float32)
    cs = jnp.dot(tril, h_ns, preferred_element_type=jnp.float32) + carry_ref[...]
    carry_ref[...] = cs[b - 1:b, :]
    h_neigh = cs / jnp.maximum(deg, 1.0)
    o_ref[...] = rb_ref[...] + jnp.dot(h_neigh, wn_ref[...],
                                       preferred_element_type=jnp.float32)


def _sc_a_body(hlo_hbm, hhi_hbm, src_hbm, dst_hbm,
               acc_out, deg_out,
               src_v, dst_v, src_t, dst_t, rows_v, rows_t,
               deg_v, acc_s, semi0, semi1, semi2, semi3,
               semg0, semg1, semg2, semg3, sems0, sems1, sems2, sems3,
               semt):
    core = lax.axis_index("c")
    sub = lax.axis_index("s")
    z16 = jnp.zeros((16,), jnp.float32)
    o16 = jnp.ones((16,), jnp.float32)


    @pl.loop(0, C)
    def _(r):
        @pl.loop(0, HDIM, step=16)
        def _(j):
            rows_v[0, r, pl.ds(j, 16)] = z16

    @pl.loop(0, NPAD, step=16)
    def _(r):
        deg_v[pl.ds(r, 16)] = z16

    rbase = sub * RPS

    @pl.loop(0, RPS, step=C)
    def _(k):
        pltpu.sync_copy(rows_v.at[0], acc_s.at[pl.ds(rbase + k, C)])

    plsc.subcore_barrier()

    ebase = sub * EPW
    nmain = EPW - CT

    semi = (semi0, semi1, semi2, semi3)
    semg = (semg0, semg1, semg2, semg3)
    sems = (sems0, sems1, sems2, sems3)

    def idx_dmas(i, b):
        yield pltpu.make_async_copy(src_hbm.at[pl.ds(ebase + i, C)],
                                    src_v.at[b], semi[b])
        yield pltpu.make_async_copy(dst_hbm.at[pl.ds(ebase + i, C)],
                                    dst_v.at[b], semi[b])

    def idx_issue(i, b):
        for d in idx_dmas(i, b):
            d.start()

    def idx_wait(i, b):
        for d in idx_dmas(i, b):
            d.wait()

    def gather_start(b):
        @pl.when(core == 0)
        def _():
            pltpu.make_async_copy(hlo_hbm.at[src_v.at[b]], rows_v.at[b],
                                  semg[b]).start()

        @pl.when(core == 1)
        def _():
            pltpu.make_async_copy(hhi_hbm.at[src_v.at[b]], rows_v.at[b],
                                  semg[b]).start()

    def gather_wait(b):
        pltpu.make_async_copy(hlo_hbm.at[src_v.at[b]], rows_v.at[b],
                              semg[b]).wait()

    def deg_update(b):
        @pl.when(core == 1)
        def _():
            for j in range(C // 16):
                idx = dst_v[b, pl.ds(j * 16, 16)]
                plsc.addupdate_scatter(deg_v, [idx], o16)

    def scatter_issue(b):
        pltpu.make_async_copy(rows_v.at[b], acc_s.at[dst_v.at[b]],
                              sems[b]).start(add=True)

    def scatter_wait(b):
        pltpu.make_async_copy(rows_v.at[b], acc_s.at[dst_v.at[b]],
                              sems[b]).wait()

    for b in range(NB):
        idx_issue(b * C, b)

    @pl.loop(0, nmain, step=NB * C)
    def _(i):
        for b in range(NB):
            idx_wait(i + b * C, b)
            gather_start(b)
            deg_update(b)
        gather_wait(0)
        scatter_issue(0)
        gather_wait(1)
        scatter_issue(1)
        scatter_wait(0)

        @pl.when(i + NB * C < nmain)
        def _():
            idx_issue(i + NB * C, 0)

        gather_wait(2)
        scatter_issue(2)
        scatter_wait(1)

        @pl.when(i + (NB + 1) * C < nmain)
        def _():
            idx_issue(i + (NB + 1) * C, 1)

        gather_wait(3)
        scatter_issue(3)
        scatter_wait(2)

        @pl.when(i + (NB + 2) * C < nmain)
        def _():
            idx_issue(i + (NB + 2) * C, 2)

        scatter_wait(3)

        @pl.when(i + (NB + 3) * C < nmain)
        def _():
            idx_issue(i + (NB + 3) * C, 3)

    bt = ebase + nmain
    pltpu.sync_copy(src_hbm.at[pl.ds(bt, CT)], src_t)
    pltpu.sync_copy(dst_hbm.at[pl.ds(bt, CT)], dst_t)

    @pl.when(core == 0)
    def _():
        pltpu.async_copy(hlo_hbm.at[src_t], rows_t, semt).wait()
        pltpu.sync_copy(rows_t, acc_s.at[dst_t], add=True)

    @pl.when(core == 1)
    def _():
        pltpu.async_copy(hhi_hbm.at[src_t], rows_t, semt).wait()
        pltpu.sync_copy(rows_t, acc_s.at[dst_t], add=True)
        for j in range(CT // 16):
            idx = dst_t[pl.ds(j * 16, 16)]
            plsc.addupdate_scatter(deg_v, [idx], o16)

    plsc.subcore_barrier()

    pltpu.sync_copy(acc_s.at[pl.ds(rbase, RPS)],
                    acc_out.at[core, pl.ds(rbase, RPS)])

    @pl.when(core == 1)
    def _():
        pltpu.sync_copy(deg_v, deg_out.at[sub])


def _sc_b_body(dst_hbm, attr_hbm, aux_out,
               dst_v, dst_t, attr_v, attr_t, aux_s,
               semi0, semi1, sems0, sems1, semt):
    core = lax.axis_index("c")
    sub = lax.axis_index("s")
    z16 = jnp.zeros((16,), jnp.float32)

    @pl.loop(0, C)
    def _(r):
        attr_v[0, r, pl.ds(0, 16)] = z16

    rbase = sub * RPS

    @pl.loop(0, RPS, step=C)
    def _(k):
        pltpu.sync_copy(attr_v.at[0], aux_s.at[pl.ds(rbase + k, C)])

    plsc.subcore_barrier()

    ebase = (core * NSUB + sub) * EPW_B
    nmain = EPW_B - CT_B

    semi = (semi0, semi1)
    sems = (sems0, sems1)

    def in_dmas(i, b):
        yield pltpu.make_async_copy(dst_hbm.at[pl.ds(ebase + i, C)],
                                    dst_v.at[b], semi[b])
        yield pltpu.make_async_copy(attr_hbm.at[pl.ds(ebase + i, C)],
                                    attr_v.at[b], semi[b])

    def in_issue(i, b):
        for d in in_dmas(i, b):
            d.start()

    def in_wait(i, b):
        for d in in_dmas(i, b):
            d.wait()

    def scatter(b, start):
        d = pltpu.make_async_copy(attr_v.at[b], aux_s.at[dst_v.at[b]],
                                  sems[b])
        if start:
            d.start(add=True)
        else:
            d.wait()

    in_issue(0, 0)
    in_issue(C, 1)

    @pl.loop(0, nmain, step=2 * C)
    def _(i):
        in_wait(i, 0)
        scatter(0, True)
        in_wait(i + C, 1)
        scatter(1, True)
        scatter(0, False)

        @pl.when(i + 2 * C < nmain)
        def _():
            in_issue(i + 2 * C, 0)

        scatter(1, False)

        @pl.when(i + 3 * C < nmain)
        def _():
            in_issue(i + 3 * C, 1)

    bt = ebase + nmain
    pltpu.sync_copy(dst_hbm.at[pl.ds(bt, CT_B)], dst_t)
    pltpu.sync_copy(attr_hbm.at[pl.ds(bt, CT_B)], attr_t)
    pltpu.sync_copy(attr_t, aux_s.at[dst_t], add=True)
    del semt

    plsc.subcore_barrier()

    pltpu.sync_copy(aux_s.at[pl.ds(rbase, RPS)],
                    aux_out.at[core, pl.ds(rbase, RPS)])


def _full(u):
    return pl.BlockSpec(u, lambda *_: tuple(0 for _ in u))


def _sc_compiler_params():
    cp = pltpu.CompilerParams(use_tc_tiling_on_sc=False)
    if "needs_layout_passes" in pltpu.CompilerParams.__dataclass_fields__:
        cp = dataclasses.replace(cp, needs_layout_passes=False)
    return cp


def kernel(x, timestamps, edge_index, edge_attr, new_node_ids,
           time_freq, time_phase, W_time, b_time,
           W_edge, b_edge, W_self, b_self, W_neigh, b_neigh):
    del new_node_ids

    f32 = jnp.float32
    npad = NPAD - N_NODES
    x_p = jnp.pad(x, ((0, npad), (0, 0)))
    ts_p = jnp.pad(timestamps, (0, npad))[:, None]
    src = edge_index[0]
    dst = edge_index[1]
    w1 = W_time[:DIM]
    w2 = W_time[DIM:]

    blk = 1024
    grid1 = NPAD // blk
    h_self = pl.pallas_call(
        _prep_body,
        grid=(grid1,),
        in_specs=[
            pl.BlockSpec((blk, DIM), lambda i: (i, 0)),
            pl.BlockSpec((blk, 1), lambda i: (i, 0)),
            _full((DIM, DIM)), _full((DIM, DIM)),
            _full((1, DIM)), _full((1, DIM)), _full((1, DIM)),
        ],
        out_specs=pl.BlockSpec((blk, DIM), lambda i: (i, 0)),
        out_shape=jax.ShapeDtypeStruct((NPAD, DIM), f32),
    )(x_p, ts_p, w1, w2, b_time[None, :], time_freq[None, :],
      time_phase[None, :])

    h_lo = h_self[:, :HDIM]
    h_hi = h_self[:, HDIM:]

    mesh = plsc.VectorSubcoreMesh(core_axis_name="c", subcore_axis_name="s")
    sc_a = pl.kernel(
        _sc_a_body,
        out_type=(
            jax.ShapeDtypeStruct((NSC, NPAD, HDIM), f32),
            jax.ShapeDtypeStruct((NSUB, NPAD), f32),
        ),
        mesh=mesh,
        compiler_params=_sc_compiler_params(),
        scratch_types=[
            pltpu.VMEM((NB, C), jnp.int32),
            pltpu.VMEM((NB, C), jnp.int32),
            pltpu.VMEM((CT,), jnp.int32),
            pltpu.VMEM((CT,), jnp.int32),
            pltpu.VMEM((NB, C, HDIM), f32),
            pltpu.VMEM((CT, HDIM), f32),
            pltpu.VMEM((NPAD,), f32),
            pltpu.VMEM_SHARED((NPAD, HDIM), f32),
        ] + [pltpu.SemaphoreType.DMA] * 13,
    )
    acc, dega = sc_a(h_lo, h_hi, src, dst)

    sc_b = pl.kernel(
        _sc_b_body,
        out_type=jax.ShapeDtypeStruct((NSC, NPAD, DE), f32),
        mesh=mesh,
        compiler_params=_sc_compiler_params(),
        scratch_types=[
            pltpu.VMEM((2, C), jnp.int32),
            pltpu.VMEM((CT_B,), jnp.int32),
            pltpu.VMEM((2, C, DE), f32),
            pltpu.VMEM((CT_B, DE), f32),
            pltpu.VMEM_SHARED((NPAD, DE), f32),
            pltpu.SemaphoreType.DMA,
            pltpu.SemaphoreType.DMA,
            pltpu.SemaphoreType.DMA,
            pltpu.SemaphoreType.DMA,
            pltpu.SemaphoreType.DMA,
        ],
    )
    aux = sc_b(dst, edge_attr)

    rst_base = pl.pallas_call(
        _rst_base_body,
        grid=(grid1,),
        in_specs=[
            pl.BlockSpec((blk, DIM), lambda i: (i, 0)),
            _full((DIM, DIM)), _full((1, DIM)), _full((1, DIM)),
        ],
        out_specs=pl.BlockSpec((blk, DIM), lambda i: (i, 0)),
        out_shape=jax.ShapeDtypeStruct((NPAD, DIM), f32),
    )(h_self, W_self, b_self[None, :], b_neigh[None, :])

    cblk = 256
    grid4 = NPAD // cblk
    rst = pl.pallas_call(
        _combine_body,
        grid=(grid4,),
        in_specs=[
            pl.BlockSpec((NSC, cblk, HDIM), lambda i: (0, i, 0)),
            pl.BlockSpec((NSC, cblk, DE), lambda i: (0, i, 0)),
            pl.BlockSpec((NSUB, cblk), lambda i: (0, i)),
            _full((NSUB, 1)),
            _full((DE, DIM)), _full((1, DIM)), _full((DIM, DIM)),
            pl.BlockSpec((cblk, DIM), lambda i: (i, 0)),
        ],
        out_specs=pl.BlockSpec((cblk, DIM), lambda i: (i, 0)),
        out_shape=jax.ShapeDtypeStruct((NPAD, DIM), f32),
        scratch_shapes=[pltpu.VMEM((1, DIM), f32)],
    )(acc, aux, dega, jnp.ones((NSUB, 1), f32), W_edge, b_edge[None, :],
      W_neigh, rst_base)

    return rst[:N_NODES]


# speedup vs baseline: 1.6081x; 1.0599x over previous
import dataclasses

import jax
import jax.numpy as jnp
from jax import lax
from jax.experimental import pallas as pl
from jax.experimental.pallas import tpu as pltpu
from jax.experimental.pallas import tpu_sc as plsc

N_NODES = 10000
NPAD = 10240
DIM = 128
DE = 16
E_TOTAL = 320000
NSC = 2
NSUB = 16
HDIM = DIM // NSC
C = 128
NB = 4
NBLK = E_TOTAL // C
BPW = NBLK // NSUB
EXTRA = NBLK - NSUB * BPW
BPW_B = NBLK // (NSC * NSUB)
EXTRA_B = NBLK - NSC * NSUB * BPW_B
RPS = NPAD // NSUB


def _prep_body(x_ref, ts_ref, w1_ref, w2_ref, bt_ref, fr_ref, ph_ref, h_ref):
    t_enc = jnp.cos(ts_ref[...] * fr_ref[...] + ph_ref[...])
    h = jnp.dot(x_ref[...], w1_ref[...], preferred_element_type=jnp.float32)
    h += jnp.dot(t_enc, w2_ref[...], preferred_element_type=jnp.float32)
    h_ref[...] = h + bt_ref[...]


def _rst_base_body(h_ref, ws_ref, bs_ref, bn_ref, o_ref):
    o_ref[...] = (jnp.dot(h_ref[...], ws_ref[...],
                          preferred_element_type=jnp.float32)
                  + bs_ref[...] + bn_ref[...])


def _combine_body(acc_ref, aux_ref, deg_ref, ones_ref, we_ref, be_ref,
                  wn_ref, rb_ref, o_ref, carry_ref):
    i = pl.program_id(0)

    @pl.when(i == 0)
    def _():
        carry_ref[...] = jnp.zeros((1, DIM), jnp.float32)

    a = jnp.concatenate([acc_ref[0], acc_ref[1]], axis=1)
    s = aux_ref[0] + aux_ref[1]
    deg = lax.dot_general(deg_ref[...], ones_ref[...],
                          (((0,), (0,)), ((), ())),
                          preferred_element_type=jnp.float32)
    h_ns = a + jnp.dot(s, we_ref[...], preferred_element_type=jnp.float32) + deg * be_ref[...]
    b = h_ns.shape[0]
    r = lax.broadcasted_iota(jnp.int32, (b, b), 0)
    c = lax.broadcasted_iota(jnp.int32, (b, b), 1)
    tril = (r >= c).astype(jnp.float32)
    cs = jnp.dot(tril, h_ns, preferred_element_type=jnp.float32) + carry_ref[...]
    carry_ref[...] = cs[b - 1:b, :]
    h_neigh = cs / jnp.maximum(deg, 1.0)
    o_ref[...] = rb_ref[...] + jnp.dot(h_neigh, wn_ref[...],
                                       preferred_element_type=jnp.float32)


def _sc_a_body(hlo_hbm, hhi_hbm, eidx_hbm,
               acc_out, deg_out,
               src_v, dst_v, rows_v,
               deg_v, acc_s, semi0, semi1, semi2, semi3,
               semg0, semg1, semg2, semg3, sems0, sems1, sems2, sems3):
    core = lax.axis_index("c")
    sub = lax.axis_index("s")
    z16 = jnp.zeros((16,), jnp.float32)
    o16 = jnp.ones((16,), jnp.float32)


    @pl.loop(0, C)
    def _(r):
        @pl.loop(0, HDIM, step=16)
        def _(j):
            rows_v[0, r, pl.ds(j, 16)] = z16

    @pl.loop(0, NPAD, step=16)
    def _(r):
        deg_v[pl.ds(r, 16)] = z16

    rbase = sub * RPS

    @pl.loop(0, RPS, step=C)
    def _(k):
        pltpu.sync_copy(rows_v.at[0], acc_s.at[pl.ds(rbase + k, C)])

    plsc.subcore_barrier()

    bbase = sub * BPW

    semi = (semi0, semi1, semi2, semi3)
    semg = (semg0, semg1, semg2, semg3)
    sems = (sems0, sems1, sems2, sems3)

    def idx_dmas(blk, b):
        yield pltpu.make_async_copy(eidx_hbm.at[blk, 0], src_v.at[b], semi[b])
        yield pltpu.make_async_copy(eidx_hbm.at[blk, 1], dst_v.at[b], semi[b])

    def idx_issue(blk, b):
        for d in idx_dmas(blk, b):
            d.start()

    def idx_wait(blk, b):
        for d in idx_dmas(blk, b):
            d.wait()

    def gather_start(b):
        @pl.when(core == 0)
        def _():
            pltpu.make_async_copy(hlo_hbm.at[src_v.at[b]], rows_v.at[b],
                                  semg[b]).start()

        @pl.when(core == 1)
        def _():
            pltpu.make_async_copy(hhi_hbm.at[src_v.at[b]], rows_v.at[b],
                                  semg[b]).start()

    def gather_wait(b):
        pltpu.make_async_copy(hlo_hbm.at[src_v.at[b]], rows_v.at[b],
                              semg[b]).wait()

    def deg_update(b):
        @pl.when(core == 1)
        def _():
            for j in range(C // 16):
                idx = dst_v[b, pl.ds(j * 16, 16)]
                plsc.addupdate_scatter(deg_v, [idx], o16)

    def scatter_issue(b):
        pltpu.make_async_copy(rows_v.at[b], acc_s.at[dst_v.at[b]],
                              sems[b]).start(add=True)

    def scatter_wait(b):
        pltpu.make_async_copy(rows_v.at[b], acc_s.at[dst_v.at[b]],
                              sems[b]).wait()

    for b in range(NB):
        idx_issue(bbase + b, b)

    @pl.loop(0, BPW, step=NB)
    def _(i):
        for b in range(NB):
            idx_wait(bbase + i + b, b)
            gather_start(b)
            deg_update(b)
        gather_wait(0)
        scatter_issue(0)
        gather_wait(1)
        scatter_issue(1)
        scatter_wait(0)

        @pl.when(i + NB < BPW)
        def _():
            idx_issue(bbase + i + NB, 0)

        gather_wait(2)
        scatter_issue(2)
        scatter_wait(1)

        @pl.when(i + NB + 1 < BPW)
        def _():
            idx_issue(bbase + i + NB + 1, 1)

        gather_wait(3)
        scatter_issue(3)
        scatter_wait(2)

        @pl.when(i + NB + 2 < BPW)
        def _():
            idx_issue(bbase + i + NB + 2, 2)

        scatter_wait(3)

        @pl.when(i + NB + 3 < BPW)
        def _():
            idx_issue(bbase + i + NB + 3, 3)

    @pl.when(sub < EXTRA)
    def _():
        blk = NSUB * BPW + sub
        idx_wait_b = list(idx_dmas(blk, 0))
        for d in idx_dmas(blk, 0):
            d.start()
        for d in idx_wait_b:
            d.wait()
        gather_start(0)
        deg_update(0)
        gather_wait(0)
        scatter_issue(0)
        scatter_wait(0)

    plsc.subcore_barrier()

    pltpu.sync_copy(acc_s.at[pl.ds(rbase, RPS)],
                    acc_out.at[core, pl.ds(rbase, RPS)])

    @pl.when(core == 1)
    def _():
        pltpu.sync_copy(deg_v, deg_out.at[sub])


def _sc_b_body(eidx_hbm, attr_hbm, aux_out,
               dst_v, attr_v, aux_s,
               semi0, semi1, sems0, sems1):
    core = lax.axis_index("c")
    sub = lax.axis_index("s")
    z16 = jnp.zeros((16,), jnp.float32)

    @pl.loop(0, C)
    def _(r):
        attr_v[0, r, pl.ds(0, 16)] = z16

    rbase = sub * RPS

    @pl.loop(0, RPS, step=C)
    def _(k):
        pltpu.sync_copy(attr_v.at[0], aux_s.at[pl.ds(rbase + k, C)])

    plsc.subcore_barrier()

    wid = core * NSUB + sub
    bbase = wid * BPW_B

    semi = (semi0, semi1)
    sems = (sems0, sems1)

    def in_dmas(blk, b):
        yield pltpu.make_async_copy(eidx_hbm.at[blk, 1], dst_v.at[b], semi[b])
        yield pltpu.make_async_copy(attr_hbm.at[pl.ds(blk * C, C)],
                                    attr_v.at[b], semi[b])

    def in_issue(blk, b):
        for d in in_dmas(blk, b):
            d.start()

    def in_wait(blk, b):
        for d in in_dmas(blk, b):
            d.wait()

    def scatter(b, start):
        d = pltpu.make_async_copy(attr_v.at[b], aux_s.at[dst_v.at[b]],
                                  sems[b])
        if start:
            d.start(add=True)
        else:
            d.wait()

    in_issue(bbase, 0)
    in_issue(bbase + 1, 1)

    @pl.loop(0, BPW_B, step=2)
    def _(i):
        in_wait(bbase + i, 0)
        scatter(0, True)
        in_wait(bbase + i + 1, 1)
        scatter(1, True)
        scatter(0, False)

        @pl.when(i + 2 < BPW_B)
        def _():
            in_issue(bbase + i + 2, 0)

        scatter(1, False)

        @pl.when(i + 3 < BPW_B)
        def _():
            in_issue(bbase + i + 3, 1)

    @pl.when(wid < EXTRA_B)
    def _():
        blk = NSC * NSUB * BPW_B + wid
        for d in in_dmas(blk, 0):
            d.start()
        for d in in_dmas(blk, 0):
            d.wait()
        scatter(0, True)
        scatter(0, False)

    plsc.subcore_barrier()

    pltpu.sync_copy(aux_s.at[pl.ds(rbase, RPS)],
                    aux_out.at[core, pl.ds(rbase, RPS)])


def _full(u):
    return pl.BlockSpec(u, lambda *_: tuple(0 for _ in u))


def _sc_compiler_params():
    cp = pltpu.CompilerParams(use_tc_tiling_on_sc=False)
    if "needs_layout_passes" in pltpu.CompilerParams.__dataclass_fields__:
        cp = dataclasses.replace(cp, needs_layout_passes=False)
    return cp


def kernel(x, timestamps, edge_index, edge_attr, new_node_ids,
           time_freq, time_phase, W_time, b_time,
           W_edge, b_edge, W_self, b_self, W_neigh, b_neigh):
    del new_node_ids

    f32 = jnp.float32
    npad = NPAD - N_NODES
    x_p = jnp.pad(x, ((0, npad), (0, 0)))
    ts_p = jnp.pad(timestamps, (0, npad))[:, None]
    eidx = jnp.transpose(jnp.reshape(edge_index, (2, NBLK, C)), (1, 0, 2))
    w1 = W_time[:DIM]
    w2 = W_time[DIM:]

    blk = 2048
    grid1 = NPAD // blk
    h_self = pl.pallas_call(
        _prep_body,
        grid=(grid1,),
        in_specs=[
            pl.BlockSpec((blk, DIM), lambda i: (i, 0)),
            pl.BlockSpec((blk, 1), lambda i: (i, 0)),
            _full((DIM, DIM)), _full((DIM, DIM)),
            _full((1, DIM)), _full((1, DIM)), _full((1, DIM)),
        ],
        out_specs=pl.BlockSpec((blk, DIM), lambda i: (i, 0)),
        out_shape=jax.ShapeDtypeStruct((NPAD, DIM), f32),
    )(x_p, ts_p, w1, w2, b_time[None, :], time_freq[None, :],
      time_phase[None, :])

    h_lo = h_self[:, :HDIM]
    h_hi = h_self[:, HDIM:]

    mesh = plsc.VectorSubcoreMesh(core_axis_name="c", subcore_axis_name="s")
    sc_a = pl.kernel(
        _sc_a_body,
        out_type=(
            jax.ShapeDtypeStruct((NSC, NPAD, HDIM), f32),
            jax.ShapeDtypeStruct((NSUB, NPAD), f32),
        ),
        mesh=mesh,
        compiler_params=_sc_compiler_params(),
        scratch_types=[
            pltpu.VMEM((NB, C), jnp.int32),
            pltpu.VMEM((NB, C), jnp.int32),
            pltpu.VMEM((NB, C, HDIM), f32),
            pltpu.VMEM((NPAD,), f32),
            pltpu.VMEM_SHARED((NPAD, HDIM), f32),
        ] + [pltpu.SemaphoreType.DMA] * 12,
    )
    acc, dega = sc_a(h_lo, h_hi, eidx)

    sc_b = pl.kernel(
        _sc_b_body,
        out_type=jax.ShapeDtypeStruct((NSC, NPAD, DE), f32),
        mesh=mesh,
        compiler_params=_sc_compiler_params(),
        scratch_types=[
            pltpu.VMEM((2, C), jnp.int32),
            pltpu.VMEM((2, C, DE), f32),
            pltpu.VMEM_SHARED((NPAD, DE), f32),
        ] + [pltpu.SemaphoreType.DMA] * 4,
    )
    aux = sc_b(eidx, edge_attr)

    rst_base = pl.pallas_call(
        _rst_base_body,
        grid=(grid1,),
        in_specs=[
            pl.BlockSpec((blk, DIM), lambda i: (i, 0)),
            _full((DIM, DIM)), _full((1, DIM)), _full((1, DIM)),
        ],
        out_specs=pl.BlockSpec((blk, DIM), lambda i: (i, 0)),
        out_shape=jax.ShapeDtypeStruct((NPAD, DIM), f32),
    )(h_self, W_self, b_self[None, :], b_neigh[None, :])

    cblk = 256
    grid4 = NPAD // cblk
    rst = pl.pallas_call(
        _combine_body,
        grid=(grid4,),
        in_specs=[
            pl.BlockSpec((NSC, cblk, HDIM), lambda i: (0, i, 0)),
            pl.BlockSpec((NSC, cblk, DE), lambda i: (0, i, 0)),
            pl.BlockSpec((NSUB, cblk), lambda i: (0, i)),
            _full((NSUB, 1)),
            _full((DE, DIM)), _full((1, DIM)), _full((DIM, DIM)),
            pl.BlockSpec((cblk, DIM), lambda i: (i, 0)),
        ],
        out_specs=pl.BlockSpec((cblk, DIM), lambda i: (i, 0)),
        out_shape=jax.ShapeDtypeStruct((NPAD, DIM), f32),
        scratch_shapes=[pltpu.VMEM((1, DIM), f32)],
    )(acc, aux, dega, jnp.ones((NSUB, 1), f32), W_edge, b_edge[None, :],
      W_neigh, rst_base)

    return rst[:N_NODES]


# speedup vs baseline: 1.7776x; 1.1054x over previous
import dataclasses

import jax
import jax.numpy as jnp
from jax import lax
from jax.experimental import pallas as pl
from jax.experimental.pallas import tpu as pltpu
from jax.experimental.pallas import tpu_sc as plsc

N_NODES = 10000
NPAD = 10240
DIM = 128
DE = 16
E_TOTAL = 320000
NSC = 2
NSUB = 16
HDIM = DIM // NSC
C = 128
NB = 4
NBLK = E_TOTAL // C
BPW = NBLK // NSUB
EXTRA = NBLK - NSUB * BPW
BPW_B = NBLK // (NSC * NSUB)
EXTRA_B = NBLK - NSC * NSUB * BPW_B
RPS = NPAD // NSUB


def _prep_body(x_ref, ts_ref, w1_ref, w2_ref, bt_ref, fr_ref, ph_ref, h_ref):
    t_enc = jnp.cos(ts_ref[...] * fr_ref[...] + ph_ref[...])
    h = jnp.dot(x_ref[...], w1_ref[...], preferred_element_type=jnp.float32)
    h += jnp.dot(t_enc, w2_ref[...], preferred_element_type=jnp.float32)
    h_ref[...] = h + bt_ref[...]


def _rst_base_body(h_ref, ws_ref, bs_ref, bn_ref, o_ref):
    o_ref[...] = (jnp.dot(h_ref[...], ws_ref[...],
                          preferred_element_type=jnp.float32)
                  + bs_ref[...] + bn_ref[...])


def _combine_body(acc_ref, aux_ref, deg_ref, ones_ref, we_ref, be_ref,
                  wn_ref, rb_ref, o_ref, carry_ref):
    i = pl.program_id(0)

    @pl.when(i == 0)
    def _():
        carry_ref[...] = jnp.zeros((1, DIM), jnp.float32)

    a = jnp.concatenate([acc_ref[0], acc_ref[1]], axis=1)
    s = aux_ref[0] + aux_ref[1]
    deg = lax.dot_general(deg_ref[...], ones_ref[...],
                          (((0,), (0,)), ((), ())),
                          preferred_element_type=jnp.float32)
    h_ns = a + jnp.dot(s, we_ref[...], preferred_element_type=jnp.float32) + deg * be_ref[...]
    b = h_ns.shape[0]
    r = lax.broadcasted_iota(jnp.int32, (b, b), 0)
    c = lax.broadcasted_iota(jnp.int32, (b, b), 1)
    tril = (r >= c).astype(jnp.float32)
    cs = jnp.dot(tril, h_ns, preferred_element_type=jnp.float32) + carry_ref[...]
    carry_ref[...] = cs[b - 1:b, :]
    h_neigh = cs / jnp.maximum(deg, 1.0)
    o_ref[...] = rb_ref[...] + jnp.dot(h_neigh, wn_ref[...],
                                       preferred_element_type=jnp.float32)


def _sc_a_body(hlo_hbm, hhi_hbm, eidx_hbm,
               acc_out, deg_out,
               src_v, dst_v, rows_v,
               deg_v, acc_s, semi0, semi1, semi2, semi3,
               semg0, semg1, semg2, semg3, sems0, sems1, sems2, sems3):
    core = lax.axis_index("c")
    sub = lax.axis_index("s")
    z16 = jnp.zeros((16,), jnp.float32)
    o16 = jnp.ones((16,), jnp.float32)


    @pl.loop(0, C)
    def _(r):
        @pl.loop(0, HDIM, step=16)
        def _(j):
            rows_v[0, r, pl.ds(j, 16)] = z16

    @pl.loop(0, NPAD, step=16)
    def _(r):
        deg_v[pl.ds(r, 16)] = z16

    rbase = sub * RPS

    @pl.loop(0, RPS, step=C)
    def _(k):
        pltpu.sync_copy(rows_v.at[0], acc_s.at[pl.ds(rbase + k, C)])

    plsc.subcore_barrier()

    bbase = sub * BPW

    semi = (semi0, semi1, semi2, semi3)
    semg = (semg0, semg1, semg2, semg3)
    sems = (sems0, sems1, sems2, sems3)

    def idx_dmas(blk, b):
        yield pltpu.make_async_copy(eidx_hbm.at[blk, 0], src_v.at[b], semi[b])
        yield pltpu.make_async_copy(eidx_hbm.at[blk, 1], dst_v.at[b], semi[b])

    def idx_issue(blk, b):
        for d in idx_dmas(blk, b):
            d.start()

    def idx_wait(blk, b):
        for d in idx_dmas(blk, b):
            d.wait()

    def gather_start(b):
        @pl.when(core == 0)
        def _():
            pltpu.make_async_copy(hlo_hbm.at[src_v.at[b]], rows_v.at[b],
                                  semg[b]).start()

        @pl.when(core == 1)
        def _():
            pltpu.make_async_copy(hhi_hbm.at[src_v.at[b]], rows_v.at[b],
                                  semg[b]).start()

    def gather_wait(b):
        pltpu.make_async_copy(hlo_hbm.at[src_v.at[b]], rows_v.at[b],
                              semg[b]).wait()

    def deg_update(b):
        @pl.when(core == 1)
        def _():
            for j in range(C // 16):
                idx = dst_v[b, pl.ds(j * 16, 16)]
                plsc.addupdate_scatter(deg_v, [idx], o16)

    def scatter_issue(b):
        pltpu.make_async_copy(rows_v.at[b], acc_s.at[dst_v.at[b]],
                              sems[b]).start(add=True)

    def scatter_wait(b):
        pltpu.make_async_copy(rows_v.at[b], acc_s.at[dst_v.at[b]],
                              sems[b]).wait()

    for b in range(NB):
        idx_issue(bbase + b, b)

    @pl.loop(0, BPW, step=NB)
    def _(i):
        for b in range(NB):
            idx_wait(bbase + i + b, b)
            gather_start(b)
            deg_update(b)
        gather_wait(0)
        scatter_issue(0)
        gather_wait(1)
        scatter_issue(1)
        scatter_wait(0)

        @pl.when(i + NB < BPW)
        def _():
            idx_issue(bbase + i + NB, 0)

        gather_wait(2)
        scatter_issue(2)
        scatter_wait(1)

        @pl.when(i + NB + 1 < BPW)
        def _():
            idx_issue(bbase + i + NB + 1, 1)

        gather_wait(3)
        scatter_issue(3)
        scatter_wait(2)

        @pl.when(i + NB + 2 < BPW)
        def _():
            idx_issue(bbase + i + NB + 2, 2)

        scatter_wait(3)

        @pl.when(i + NB + 3 < BPW)
        def _():
            idx_issue(bbase + i + NB + 3, 3)

    @pl.when(sub < EXTRA)
    def _():
        blk = NSUB * BPW + sub
        idx_wait_b = list(idx_dmas(blk, 0))
        for d in idx_dmas(blk, 0):
            d.start()
        for d in idx_wait_b:
            d.wait()
        gather_start(0)
        deg_update(0)
        gather_wait(0)
        scatter_issue(0)
        scatter_wait(0)

    plsc.subcore_barrier()

    pltpu.sync_copy(acc_s.at[pl.ds(rbase, RPS)],
                    acc_out.at[core, pl.ds(rbase, RPS)])

    @pl.when(core == 1)
    def _():
        pltpu.sync_copy(deg_v, deg_out.at[sub])


def _sc_b_body(eidx_hbm, attr_hbm, aux_out,
               dst_v, attr_v, aux_s,
               semi0, semi1, semi2, semi3, sems0, sems1, sems2, sems3):
    core = lax.axis_index("c")
    sub = lax.axis_index("s")
    z16 = jnp.zeros((16,), jnp.float32)

    @pl.loop(0, C)
    def _(r):
        attr_v[0, r, pl.ds(0, 16)] = z16

    rbase = sub * RPS

    @pl.loop(0, RPS, step=C)
    def _(k):
        pltpu.sync_copy(attr_v.at[0], aux_s.at[pl.ds(rbase + k, C)])

    plsc.subcore_barrier()

    wid = core * NSUB + sub
    bbase = wid * BPW_B

    semi = (semi0, semi1, semi2, semi3)
    sems = (sems0, sems1, sems2, sems3)

    def in_dmas(blk, b):
        yield pltpu.make_async_copy(eidx_hbm.at[blk, 1], dst_v.at[b], semi[b])
        yield pltpu.make_async_copy(attr_hbm.at[pl.ds(blk * C, C)],
                                    attr_v.at[b], semi[b])

    def in_issue(blk, b):
        for d in in_dmas(blk, b):
            d.start()

    def in_wait(blk, b):
        for d in in_dmas(blk, b):
            d.wait()

    def scatter(b, start):
        d = pltpu.make_async_copy(attr_v.at[b], aux_s.at[dst_v.at[b]],
                                  sems[b])
        if start:
            d.start(add=True)
        else:
            d.wait()

    for b in range(NB):
        in_issue(bbase + b, b)

    @pl.loop(0, BPW_B - 2, step=NB)
    def _(i):
        for b in range(NB):
            in_wait(bbase + i + b, b)
            scatter(b, True)
        for b in range(NB):
            scatter(b, False)

            @pl.when(i + NB + b < BPW_B)
            def _():
                in_issue(bbase + i + NB + b, b)

    for b in range(2):
        in_wait(bbase + BPW_B - 2 + b, b)
        scatter(b, True)
    for b in range(2):
        scatter(b, False)

    @pl.when(wid < EXTRA_B)
    def _():
        blk = NSC * NSUB * BPW_B + wid
        for d in in_dmas(blk, 0):
            d.start()
        for d in in_dmas(blk, 0):
            d.wait()
        scatter(0, True)
        scatter(0, False)

    plsc.subcore_barrier()

    pltpu.sync_copy(aux_s.at[pl.ds(rbase, RPS)],
                    aux_out.at[core, pl.ds(rbase, RPS)])


def _full(u):
    return pl.BlockSpec(u, lambda *_: tuple(0 for _ in u))


def _sc_compiler_params():
    cp = pltpu.CompilerParams(use_tc_tiling_on_sc=False)
    if "needs_layout_passes" in pltpu.CompilerParams.__dataclass_fields__:
        cp = dataclasses.replace(cp, needs_layout_passes=False)
    return cp


def kernel(x, timestamps, edge_index, edge_attr, new_node_ids,
           time_freq, time_phase, W_time, b_time,
           W_edge, b_edge, W_self, b_self, W_neigh, b_neigh):
    del new_node_ids

    f32 = jnp.float32
    npad = NPAD - N_NODES
    x_p = jnp.pad(x, ((0, npad), (0, 0)))
    ts_p = jnp.pad(timestamps, (0, npad))[:, None]
    eidx = jnp.transpose(jnp.reshape(edge_index, (2, NBLK, C)), (1, 0, 2))
    w1 = W_time[:DIM]
    w2 = W_time[DIM:]

    blk = 2048
    grid1 = NPAD // blk
    h_self = pl.pallas_call(
        _prep_body,
        grid=(grid1,),
        in_specs=[
            pl.BlockSpec((blk, DIM), lambda i: (i, 0)),
            pl.BlockSpec((blk, 1), lambda i: (i, 0)),
            _full((DIM, DIM)), _full((DIM, DIM)),
            _full((1, DIM)), _full((1, DIM)), _full((1, DIM)),
        ],
        out_specs=pl.BlockSpec((blk, DIM), lambda i: (i, 0)),
        out_shape=jax.ShapeDtypeStruct((NPAD, DIM), f32),
    )(x_p, ts_p, w1, w2, b_time[None, :], time_freq[None, :],
      time_phase[None, :])

    h_lo = h_self[:, :HDIM]
    h_hi = h_self[:, HDIM:]

    mesh = plsc.VectorSubcoreMesh(core_axis_name="c", subcore_axis_name="s")
    sc_a = pl.kernel(
        _sc_a_body,
        out_type=(
            jax.ShapeDtypeStruct((NSC, NPAD, HDIM), f32),
            jax.ShapeDtypeStruct((NSUB, NPAD), f32),
        ),
        mesh=mesh,
        compiler_params=_sc_compiler_params(),
        scratch_types=[
            pltpu.VMEM((NB, C), jnp.int32),
            pltpu.VMEM((NB, C), jnp.int32),
            pltpu.VMEM((NB, C, HDIM), f32),
            pltpu.VMEM((NPAD,), f32),
            pltpu.VMEM_SHARED((NPAD, HDIM), f32),
        ] + [pltpu.SemaphoreType.DMA] * 12,
    )
    acc, dega = sc_a(h_lo, h_hi, eidx)

    sc_b = pl.kernel(
        _sc_b_body,
        out_type=jax.ShapeDtypeStruct((NSC, NPAD, DE), f32),
        mesh=mesh,
        compiler_params=_sc_compiler_params(),
        scratch_types=[
            pltpu.VMEM((NB, C), jnp.int32),
            pltpu.VMEM((NB, C, DE), f32),
            pltpu.VMEM_SHARED((NPAD, DE), f32),
        ] + [pltpu.SemaphoreType.DMA] * 8,
    )
    aux = sc_b(eidx, edge_attr)

    rst_base = pl.pallas_call(
        _rst_base_body,
        grid=(grid1,),
        in_specs=[
            pl.BlockSpec((blk, DIM), lambda i: (i, 0)),
            _full((DIM, DIM)), _full((1, DIM)), _full((1, DIM)),
        ],
        out_specs=pl.BlockSpec((blk, DIM), lambda i: (i, 0)),
        out_shape=jax.ShapeDtypeStruct((NPAD, DIM), f32),
    )(h_self, W_self, b_self[None, :], b_neigh[None, :])

    cblk = 512
    grid4 = NPAD // cblk
    rst = pl.pallas_call(
        _combine_body,
        grid=(grid4,),
        in_specs=[
            pl.BlockSpec((NSC, cblk, HDIM), lambda i: (0, i, 0)),
            pl.BlockSpec((NSC, cblk, DE), lambda i: (0, i, 0)),
            pl.BlockSpec((NSUB, cblk), lambda i: (0, i)),
            _full((NSUB, 1)),
            _full((DE, DIM)), _full((1, DIM)), _full((DIM, DIM)),
            pl.BlockSpec((cblk, DIM), lambda i: (i, 0)),
        ],
        out_specs=pl.BlockSpec((cblk, DIM), lambda i: (i, 0)),
        out_shape=jax.ShapeDtypeStruct((NPAD, DIM), f32),
        scratch_shapes=[pltpu.VMEM((1, DIM), f32)],
    )(acc, aux, dega, jnp.ones((NSUB, 1), f32), W_edge, b_edge[None, :],
      W_neigh, rst_base)

    return rst[:N_NODES]
